# Initial kernel scaffold; baseline (speedup 1.0000x reference)
import jax
import jax.numpy as jnp
from jax.experimental import pallas as pl


def kernel(x, embedding):
    raise NotImplementedError("write your pallas kernel here")


# speedup vs baseline: 1.5401x; 1.5401x over previous
import functools
import jax
import jax.numpy as jnp
from jax import lax
from jax.experimental import pallas as pl
from jax.experimental.pallas import tpu as pltpu
from jax.experimental.pallas import tpu_sc as plsc


def _build(B, V, D, dtype):
    info = plsc.get_sparse_core_info()
    NC, NS = info.num_cores, info.num_subcores
    NW = NC * NS
    assert B % NW == 0
    b_per_w = B // NW
    chunk = 64
    assert b_per_w % chunk == 0
    nchunk = b_per_w // chunk

    mesh = plsc.VectorSubcoreMesh(core_axis_name="c", subcore_axis_name="s")

    @functools.partial(
        pl.kernel,
        mesh=mesh,
        out_type=jax.ShapeDtypeStruct((B, D), dtype),
        scratch_types=[
            pltpu.VMEM((b_per_w,), jnp.int32),
            pltpu.VMEM((chunk, D), dtype),
            pltpu.SemaphoreType.DMA,
        ],
    )
    def embed(idx_hbm, table_hbm, out_hbm, idx_v, rows_v, gsem):
        wid = lax.axis_index("s") * NC + lax.axis_index("c")
        base = wid * b_per_w
        pltpu.sync_copy(idx_hbm.at[pl.ds(base, b_per_w)], idx_v)
        for i in range(nchunk):
            pltpu.async_copy(
                table_hbm.at[idx_v.at[pl.ds(i * chunk, chunk)]],
                rows_v,
                gsem,
            ).wait()
            pltpu.sync_copy(rows_v, out_hbm.at[pl.ds(base + i * chunk, chunk)])

    return embed


def kernel(x, embedding):
    B_, S_ = x.shape
    V, D = embedding.shape
    B = B_ * S_
    idx = x.reshape(B).astype(jnp.int32)
    embed = _build(B, V, D, embedding.dtype)
    out = embed(idx, embedding)
    return out.reshape(B_, S_, D)


# speedup vs baseline: 1.5720x; 1.0207x over previous
import functools
import jax
import jax.numpy as jnp
from jax import lax
from jax.experimental import pallas as pl
from jax.experimental.pallas import tpu as pltpu
from jax.experimental.pallas import tpu_sc as plsc


def _build(B, V, D, dtype):
    info = plsc.get_sparse_core_info()
    NC, NS = info.num_cores, info.num_subcores
    NW = NC * NS
    assert B % NW == 0
    b_per_w = B // NW
    chunk = 32
    nbuf = 3
    assert b_per_w % chunk == 0
    nchunk = b_per_w // chunk

    mesh = plsc.VectorSubcoreMesh(core_axis_name="c", subcore_axis_name="s")

    @functools.partial(
        pl.kernel,
        mesh=mesh,
        out_type=jax.ShapeDtypeStruct((B, D), dtype),
        scratch_types=[
            pltpu.VMEM((b_per_w,), jnp.int32),
            pltpu.VMEM((nbuf, chunk, D), dtype),
        ]
        + [pltpu.SemaphoreType.DMA] * (2 * nbuf),
    )
    def embed(idx_hbm, table_hbm, out_hbm, idx_v, rows_v, *sems):
        gsem = sems[:nbuf]
        ssem = sems[nbuf:]
        wid = lax.axis_index("s") * NC + lax.axis_index("c")
        base = wid * b_per_w
        pltpu.sync_copy(idx_hbm.at[pl.ds(base, b_per_w)], idx_v)

        gathers = [None] * nbuf
        scatters = [None] * nbuf

        def start_gather(i):
            b = i % nbuf
            gathers[b] = pltpu.async_copy(
                table_hbm.at[idx_v.at[pl.ds(i * chunk, chunk)]],
                rows_v.at[b],
                gsem[b],
            )

        start_gather(0)
        for i in range(nchunk):
            b = i % nbuf
            gathers[b].wait()
            scatters[b] = pltpu.async_copy(
                rows_v.at[b],
                out_hbm.at[pl.ds(base + i * chunk, chunk)],
                ssem[b],
            )
            j = i + 1
            if j < nchunk:
                bj = j % nbuf
                if scatters[bj] is not None:
                    scatters[bj].wait()
                start_gather(j)
        for b in range(nbuf):
            if scatters[b] is not None:
                scatters[b].wait()

    return embed


def kernel(x, embedding):
    B_, S_ = x.shape
    V, D = embedding.shape
    B = B_ * S_
    idx = x.reshape(B).astype(jnp.int32)
    embed = _build(B, V, D, embedding.dtype)
    out = embed(idx, embedding)
    return out.reshape(B_, S_, D)
